# baseline (speedup 1.0000x reference)
import jax
import jax.numpy as jnp
from jax import lax
from jax.experimental import pallas as pl
from jax.experimental.pallas import tpu as pltpu
from jax.experimental.pallas import tpu_sc as plsc

_EMB = 32
_PACK = 128 // _EMB
_BATCH = 16384
_NC = 2
_NS = 16
_NW = _NC * _NS
_BPW = _BATCH // _NW
_CHUNK = 128
_NCHUNK = _BPW // _CHUNK
_LANES = 16


def _sc_gather_body(gid_hbm, aid_hbm, rid_hbm,
                    g128_hbm, a128_hbm, r128_hbm,
                    gout_hbm, aout_hbm, rout_hbm,
                    idx_v, idx4_v, dst_v, out_v, sem):
    wid = lax.axis_index("s") * _NC + lax.axis_index("c")
    base = wid * _BPW

    for ids, t128, out in ((gid_hbm, g128_hbm, gout_hbm),
                           (aid_hbm, a128_hbm, aout_hbm),
                           (rid_hbm, r128_hbm, rout_hbm)):
        pltpu.sync_copy(ids.at[pl.ds(base, _BPW)], idx_v)

        def _shift(c):
            idx4_v[pl.ds(c * _LANES, _LANES)] = (
                idx_v[pl.ds(c * _LANES, _LANES)] >> 2)

        pl.loop(0, _BPW // _LANES)(_shift)

        for h in range(2):
            hbase = h * 2 * _CHUNK
            copies = [
                pltpu.async_copy(
                    t128.at[idx4_v.at[pl.ds(hbase + k * _CHUNK, _CHUNK)]],
                    dst_v.at[pl.ds(k * _CHUNK, _CHUNK)],
                    sem)
                for k in range(2)
            ]
            for cp in copies:
                cp.wait()

            def _extract(c, hbase=hbase):
                vec = idx_v[pl.ds(hbase + c * _LANES, _LANES)]
                off = (vec & 3) << 5
                rows = lax.iota(jnp.int32, _LANES) + c * _LANES
                for j in range(_EMB):
                    x = plsc.load_gather(dst_v, [rows, off + j])
                    plsc.store_scatter(
                        out_v,
                        [rows + hbase, jnp.full((_LANES,), j, jnp.int32)], x)

            pl.loop(0, 2 * _CHUNK // _LANES)(_extract)

        pltpu.sync_copy(out_v, out.at[pl.ds(base, _BPW)])


_sc_gather = pl.kernel(
    _sc_gather_body,
    out_type=(
        jax.ShapeDtypeStruct((_BATCH, _EMB), jnp.float32),
        jax.ShapeDtypeStruct((_BATCH, _EMB), jnp.float32),
        jax.ShapeDtypeStruct((_BATCH, _EMB), jnp.float32),
    ),
    mesh=plsc.VectorSubcoreMesh(core_axis_name="c", subcore_axis_name="s"),
    compiler_params=pltpu.CompilerParams(needs_layout_passes=False),
    scratch_types=[
        pltpu.VMEM((_BPW,), jnp.int32),
        pltpu.VMEM((_BPW,), jnp.int32),
        pltpu.VMEM((2 * _CHUNK, 128), jnp.float32),
        pltpu.VMEM((_BPW, _EMB), jnp.float32),
        pltpu.SemaphoreType.DMA,
    ],
)


_BLK = 2048


def _mlp_body(g_ref, a_ref, r_ref, W1_ref, b1_ref, W2_ref, b2_ref,
              W3_ref, b3_ref, out_ref):
    w1 = W1_ref[...]
    h = jnp.dot(g_ref[...], w1[0:_EMB, :], preferred_element_type=jnp.float32)
    h += jnp.dot(a_ref[...], w1[_EMB:2 * _EMB, :],
                 preferred_element_type=jnp.float32)
    h += jnp.dot(r_ref[...], w1[2 * _EMB:3 * _EMB, :],
                 preferred_element_type=jnp.float32)
    h = jnp.maximum(h + b1_ref[...], 0.0)
    h = jnp.dot(h, W2_ref[...], preferred_element_type=jnp.float32)
    h = jnp.maximum(h + b2_ref[...], 0.0)
    out_ref[...] = (jnp.dot(h, W3_ref[...], preferred_element_type=jnp.float32)
                    + b3_ref[...])


def _mlp(g, a, r, W1, b1, W2, b2, W3, b3):
    grid = (_BATCH // _BLK,)
    row_spec = pl.BlockSpec((_BLK, _EMB), lambda i: (i, 0))
    full = lambda shape: pl.BlockSpec(shape, lambda i: (0,) * len(shape))
    return pl.pallas_call(
        _mlp_body,
        grid=grid,
        in_specs=[
            row_spec, row_spec, row_spec,
            full((3 * _EMB, 64)), full((1, 64)),
            full((64, _EMB)), full((1, _EMB)),
            full((_EMB, _EMB)), full((1, _EMB)),
        ],
        out_specs=pl.BlockSpec((_BLK, _EMB), lambda i: (i, 0)),
        out_shape=jax.ShapeDtypeStruct((_BATCH, _EMB), jnp.float32),
    )(g, a, r, W1, b1.reshape(1, -1), W2, b2.reshape(1, -1),
      W3, b3.reshape(1, -1))


def kernel(genre_id, author_id, artist_id, genre_table, author_table,
           artist_table, W1, b1, W2, b2, W3, b3):
    g128 = genre_table.reshape(-1, 128)
    a128 = author_table.reshape(-1, 128)
    r128 = artist_table.reshape(-1, 128)
    g, a, r = _sc_gather(genre_id, author_id, artist_id, g128, a128, r128)
    return _mlp(g, a, r, W1, b1, W2, b2, W3, b3)


# speedup vs baseline: 1.5406x; 1.5406x over previous
import jax
import jax.numpy as jnp
from jax import lax
from jax.experimental import pallas as pl
from jax.experimental.pallas import tpu as pltpu
from jax.experimental.pallas import tpu_sc as plsc

_EMB = 32
_PACK = 128 // _EMB
_BATCH = 16384
_NC = 2
_NS = 16
_NW = _NC * _NS
_BPW = _BATCH // _NW
_CHUNK = 128
_NCHUNK = _BPW // _CHUNK
_LANES = 16


def _sc_gather_body(gid_hbm, aid_hbm, rid_hbm,
                    g128_hbm, a128_hbm, r128_hbm,
                    gout_hbm, aout_hbm, rout_hbm,
                    idx_v, idx4_v, dst_v, out_v, sem):
    wid = lax.axis_index("s") * _NC + lax.axis_index("c")
    base = wid * _BPW

    for ids, t128, out in ((gid_hbm, g128_hbm, gout_hbm),
                           (aid_hbm, a128_hbm, aout_hbm),
                           (rid_hbm, r128_hbm, rout_hbm)):
        pltpu.sync_copy(ids.at[pl.ds(base, _BPW)], idx_v)

        def _shift(c):
            idx4_v[pl.ds(c * _LANES, _LANES)] = (
                idx_v[pl.ds(c * _LANES, _LANES)] >> 2)

        pl.loop(0, _BPW // _LANES)(_shift)

        for h in range(2):
            hbase = h * 2 * _CHUNK
            copies = [
                pltpu.async_copy(
                    t128.at[idx4_v.at[pl.ds(hbase + k * _CHUNK, _CHUNK)]],
                    dst_v.at[pl.ds(k * _CHUNK, _CHUNK)],
                    sem)
                for k in range(2)
            ]
            for cp in copies:
                cp.wait()

            def _extract(c, hbase=hbase):
                vec = idx_v[pl.ds(hbase + c * _LANES, _LANES)]
                off = (vec & 3) << 5
                rows = lax.iota(jnp.int32, _LANES) + c * _LANES
                jcol = lax.iota(jnp.int32, _LANES)
                xs = [plsc.load_gather(dst_v, [rows, off + j])
                      for j in range(_EMB)]
                for j in range(_EMB):
                    plsc.store_scatter(out_v, [rows + hbase, jcol * 0 + j],
                                       xs[j])

            pl.loop(0, 2 * _CHUNK // _LANES)(_extract)

        pltpu.sync_copy(out_v, out.at[pl.ds(base, _BPW)])


_sc_gather = pl.kernel(
    _sc_gather_body,
    out_type=(
        jax.ShapeDtypeStruct((_BATCH, _EMB), jnp.float32),
        jax.ShapeDtypeStruct((_BATCH, _EMB), jnp.float32),
        jax.ShapeDtypeStruct((_BATCH, _EMB), jnp.float32),
    ),
    mesh=plsc.VectorSubcoreMesh(core_axis_name="c", subcore_axis_name="s"),
    compiler_params=pltpu.CompilerParams(needs_layout_passes=False),
    scratch_types=[
        pltpu.VMEM((_BPW,), jnp.int32),
        pltpu.VMEM((_BPW,), jnp.int32),
        pltpu.VMEM((2 * _CHUNK, 128), jnp.float32),
        pltpu.VMEM((_BPW, _EMB), jnp.float32),
        pltpu.SemaphoreType.DMA,
    ],
)


_GRP = 4


def _pack_shuffle(src_v, s0, width, out_v):
    lanes = lax.iota(jnp.int32, _LANES)
    js = [(lanes + h * _LANES) & 31 for h in range(2)]
    ks = [(lanes + h * _LANES) >> 5 for h in range(2)]
    for q0 in range(0, width // 4, 8):
        nq = min(8, width // 4 - q0)
        xs = [plsc.load_gather(src_v, [js[h], s0 + 4 * (q0 + q) + ks[h]])
              for q in range(nq) for h in range(2)]
        i = 0
        for q in range(nq):
            for h in range(2):
                out_v[q0 + q, pl.ds(h * _LANES, _LANES)] = xs[i]
                i += 1


def _sc_pack_body(gT_hbm, aT_hbm, rT_hbm,
                  gtail_hbm, atail_hbm, rtail_hbm,
                  g128_hbm, a128_hbm, r128_hbm,
                  in_v, out_v, sem):
    wid = lax.axis_index("s") * _NC + lax.axis_index("c")

    for tT, tail, t128, n in ((gT_hbm, gtail_hbm, g128_hbm, 1000),
                              (aT_hbm, atail_hbm, a128_hbm, 100000),
                              (rT_hbm, rtail_hbm, r128_hbm, 1000000)):
        full = n // 128
        ngrp = (full + _GRP - 1) // _GRP
        gw = (ngrp + _NW - 1) // _NW
        lo = jnp.minimum(wid * gw, ngrp)
        hi = jnp.minimum(lo + gw, ngrp)

        def _group(g, tT=tT, t128=t128, full=full):
            c0 = jnp.minimum(g * _GRP, full - _GRP)
            pltpu.sync_copy(
                tT.at[:, pl.ds(pl.multiple_of(c0 * 128, 128), _GRP * 128)],
                in_v)

            def _slab(s):
                _pack_shuffle(in_v, s * 128, 128,
                              out_v.at[pl.ds(pl.multiple_of(s * 32, 32),
                                             32)])

            pl.loop(0, _GRP)(_slab)
            pltpu.sync_copy(
                out_v, t128.at[pl.ds(pl.multiple_of(c0 * 32, 32),
                                     _GRP * 32)])

        lax.fori_loop(lo, hi, lambda g, c: (_group(g), c)[1], 0)

        @pl.when(wid == _NW - 1)
        def _tail(tail=tail, t128=t128, full=full, n=n):
            pltpu.sync_copy(tail, t128.at[pl.ds(full * 32, (n % 128) // 4)])


_sc_pack = pl.kernel(
    _sc_pack_body,
    out_type=(
        jax.ShapeDtypeStruct((250, 128), jnp.float32),
        jax.ShapeDtypeStruct((25000, 128), jnp.float32),
        jax.ShapeDtypeStruct((250000, 128), jnp.float32),
    ),
    mesh=plsc.VectorSubcoreMesh(core_axis_name="c", subcore_axis_name="s"),
    compiler_params=pltpu.CompilerParams(needs_layout_passes=False),
    scratch_types=[
        pltpu.VMEM((_EMB, _GRP * 128), jnp.float32),
        pltpu.VMEM((_GRP * 32, 128), jnp.float32),
        pltpu.SemaphoreType.DMA,
    ],
)


_BLK = 2048


def _mlp_body(g_ref, a_ref, r_ref, W1_ref, b1_ref, W2_ref, b2_ref,
              W3_ref, b3_ref, out_ref):
    w1 = W1_ref[...]
    h = jnp.dot(g_ref[...], w1[0:_EMB, :], preferred_element_type=jnp.float32)
    h += jnp.dot(a_ref[...], w1[_EMB:2 * _EMB, :],
                 preferred_element_type=jnp.float32)
    h += jnp.dot(r_ref[...], w1[2 * _EMB:3 * _EMB, :],
                 preferred_element_type=jnp.float32)
    h = jnp.maximum(h + b1_ref[...], 0.0)
    h = jnp.dot(h, W2_ref[...], preferred_element_type=jnp.float32)
    h = jnp.maximum(h + b2_ref[...], 0.0)
    out_ref[...] = (jnp.dot(h, W3_ref[...], preferred_element_type=jnp.float32)
                    + b3_ref[...])


def _mlp(g, a, r, W1, b1, W2, b2, W3, b3):
    grid = (_BATCH // _BLK,)
    row_spec = pl.BlockSpec((_BLK, _EMB), lambda i: (i, 0))
    full = lambda shape: pl.BlockSpec(shape, lambda i: (0,) * len(shape))
    return pl.pallas_call(
        _mlp_body,
        grid=grid,
        in_specs=[
            row_spec, row_spec, row_spec,
            full((3 * _EMB, 64)), full((1, 64)),
            full((64, _EMB)), full((1, _EMB)),
            full((_EMB, _EMB)), full((1, _EMB)),
        ],
        out_specs=pl.BlockSpec((_BLK, _EMB), lambda i: (i, 0)),
        out_shape=jax.ShapeDtypeStruct((_BATCH, _EMB), jnp.float32),
    )(g, a, r, W1, b1.reshape(1, -1), W2, b2.reshape(1, -1),
      W3, b3.reshape(1, -1))


def kernel(genre_id, author_id, artist_id, genre_table, author_table,
           artist_table, W1, b1, W2, b2, W3, b3):
    def _tail128(t):
        return t[(t.shape[0] // 128) * 128:].reshape(-1, 128)

    g128, a128, r128 = _sc_pack(genre_table.T, author_table.T,
                                artist_table.T, _tail128(genre_table),
                                _tail128(author_table),
                                _tail128(artist_table))
    g, a, r = _sc_gather(genre_id, author_id, artist_id, g128, a128, r128)
    return _mlp(g, a, r, W1, b1, W2, b2, W3, b3)


# speedup vs baseline: 1.9179x; 1.2449x over previous
import jax
import jax.numpy as jnp
from jax import lax
from jax.experimental import pallas as pl
from jax.experimental.pallas import tpu as pltpu
from jax.experimental.pallas import tpu_sc as plsc

_EMB = 32
_PACK = 128 // _EMB
_BATCH = 16384
_NC = 2
_NS = 16
_NW = _NC * _NS
_BPW = _BATCH // _NW
_CHUNK = 128
_NCHUNK = _BPW // _CHUNK
_LANES = 16


def _sc_gather_body(gid_hbm, aid_hbm, rid_hbm,
                    g128_hbm, a128_hbm, r128_hbm,
                    gout_hbm, aout_hbm, rout_hbm,
                    idx_v, idx4_v, dst_v, out_v, sem):
    wid = lax.axis_index("s") * _NC + lax.axis_index("c")
    base = wid * _BPW

    for ids, t128, out in ((gid_hbm, g128_hbm, gout_hbm),
                           (aid_hbm, a128_hbm, aout_hbm),
                           (rid_hbm, r128_hbm, rout_hbm)):
        pltpu.sync_copy(ids.at[pl.ds(base, _BPW)], idx_v)

        def _shift(c):
            idx4_v[pl.ds(c * _LANES, _LANES)] = (
                idx_v[pl.ds(c * _LANES, _LANES)] >> 2)

        pl.loop(0, _BPW // _LANES)(_shift)

        for h in range(2):
            hbase = h * 2 * _CHUNK
            copies = [
                pltpu.async_copy(
                    t128.at[idx4_v.at[pl.ds(hbase + k * _CHUNK, _CHUNK)]],
                    dst_v.at[pl.ds(k * _CHUNK, _CHUNK)],
                    sem)
                for k in range(2)
            ]
            for cp in copies:
                cp.wait()

            def _extract(c, hbase=hbase):
                vec = idx_v[pl.ds(hbase + c * _LANES, _LANES)]
                off = (vec & 3) << 5
                rows = lax.iota(jnp.int32, _LANES) + c * _LANES
                jcol = lax.iota(jnp.int32, _LANES)
                xs = [plsc.load_gather(dst_v, [rows, off + j])
                      for j in range(_EMB)]
                for j in range(_EMB):
                    plsc.store_scatter(out_v, [rows + hbase, jcol * 0 + j],
                                       xs[j])

            pl.loop(0, 2 * _CHUNK // _LANES)(_extract)

        pltpu.sync_copy(out_v, out.at[pl.ds(base, _BPW)])


_sc_gather = pl.kernel(
    _sc_gather_body,
    out_type=(
        jax.ShapeDtypeStruct((_BATCH, _EMB), jnp.float32),
        jax.ShapeDtypeStruct((_BATCH, _EMB), jnp.float32),
        jax.ShapeDtypeStruct((_BATCH, _EMB), jnp.float32),
    ),
    mesh=plsc.VectorSubcoreMesh(core_axis_name="c", subcore_axis_name="s"),
    compiler_params=pltpu.CompilerParams(needs_layout_passes=False),
    scratch_types=[
        pltpu.VMEM((_BPW,), jnp.int32),
        pltpu.VMEM((_BPW,), jnp.int32),
        pltpu.VMEM((2 * _CHUNK, 128), jnp.float32),
        pltpu.VMEM((_BPW, _EMB), jnp.float32),
        pltpu.SemaphoreType.DMA,
    ],
)


_GRP = 4


def _pack_shuffle(src_v, s0, width, out_v):
    lanes = lax.iota(jnp.int32, _LANES)
    js = [(lanes + h * _LANES) & 31 for h in range(2)]
    ks = [(lanes + h * _LANES) >> 5 for h in range(2)]
    for q0 in range(0, width // 4, 8):
        nq = min(8, width // 4 - q0)
        xs = [plsc.load_gather(src_v, [js[h], s0 + 4 * (q0 + q) + ks[h]])
              for q in range(nq) for h in range(2)]
        i = 0
        for q in range(nq):
            for h in range(2):
                out_v[q0 + q, pl.ds(h * _LANES, _LANES)] = xs[i]
                i += 1


def _sc_pack_body(gT_hbm, aT_hbm, rT_hbm,
                  gtail_hbm, atail_hbm, rtail_hbm,
                  g128_hbm, a128_hbm, r128_hbm,
                  in_v, inb_v, out_v, sem, semb):
    wid = lax.axis_index("s") * _NC + lax.axis_index("c")

    for tT, tail, t128, n, grp in ((gT_hbm, gtail_hbm, g128_hbm, 1000, 4),
                                   (aT_hbm, atail_hbm, a128_hbm, 100000,
                                    _GRP),
                                   (rT_hbm, rtail_hbm, r128_hbm, 1000000,
                                    _GRP)):
        full = n // 128
        ngrp = (full + grp - 1) // grp
        gw = (ngrp + _NW - 1) // _NW
        lo = jnp.minimum(wid * gw, ngrp)
        hi = jnp.minimum(lo + gw, ngrp)

        def _slabs(g, tT=tT, full=full, grp=grp):
            c0 = jnp.minimum(g * grp, full - grp)
            return tT.at[:, pl.ds(pl.multiple_of(c0 * 128, 128),
                                  grp * 128)]

        def _proc(g, buf, t128=t128, full=full, grp=grp):
            c0 = jnp.minimum(g * grp, full - grp)
            for s in range(grp):
                _pack_shuffle(buf, s * 128, 128,
                              out_v.at[pl.ds(s * 32, 32)])
            pltpu.sync_copy(
                out_v, t128.at[pl.ds(pl.multiple_of(c0 * 32, 32),
                                     grp * 32)])

        if n < 1000000:
            def _group(g, tT=tT):
                pltpu.sync_copy(_slabs(g), in_v)
                _proc(g, in_v)

            lax.fori_loop(lo, hi, lambda g, c: (_group(g), c)[1], 0)
        else:
            drainA = lambda: pltpu.make_async_copy(
                tT.at[:, pl.ds(0, grp * 128)], in_v, sem).wait()
            drainB = lambda: pltpu.make_async_copy(
                tT.at[:, pl.ds(0, grp * 128)], inb_v, semb).wait()
            pltpu.async_copy(_slabs(lo), in_v, sem)

            def _pair(i, c):
                g0 = lo + 2 * i
                pltpu.async_copy(_slabs(g0 + 1), inb_v, semb)
                drainA()
                _proc(g0, in_v)
                pltpu.async_copy(_slabs(g0 + 2), in_v, sem)
                drainB()
                _proc(g0 + 1, inb_v)
                return c

            lax.fori_loop(0, (hi - lo + 1) // 2, _pair, 0)
            drainA()

        @pl.when(wid == _NW - 1)
        def _tail(tail=tail, t128=t128, full=full, n=n):
            pltpu.sync_copy(tail, t128.at[pl.ds(full * 32, (n % 128) // 4)])


_sc_pack = pl.kernel(
    _sc_pack_body,
    out_type=(
        jax.ShapeDtypeStruct((250, 128), jnp.float32),
        jax.ShapeDtypeStruct((25000, 128), jnp.float32),
        jax.ShapeDtypeStruct((250000, 128), jnp.float32),
    ),
    mesh=plsc.VectorSubcoreMesh(core_axis_name="c", subcore_axis_name="s"),
    compiler_params=pltpu.CompilerParams(needs_layout_passes=False),
    scratch_types=[
        pltpu.VMEM((_EMB, _GRP * 128), jnp.float32),
        pltpu.VMEM((_EMB, _GRP * 128), jnp.float32),
        pltpu.VMEM((_GRP * 32, 128), jnp.float32),
        pltpu.SemaphoreType.DMA,
        pltpu.SemaphoreType.DMA,
    ],
)


_BLK = 2048


def _mlp_body(g_ref, a_ref, r_ref, W1_ref, b1_ref, W2_ref, b2_ref,
              W3_ref, b3_ref, out_ref):
    w1 = W1_ref[...]
    h = jnp.dot(g_ref[...], w1[0:_EMB, :], preferred_element_type=jnp.float32)
    h += jnp.dot(a_ref[...], w1[_EMB:2 * _EMB, :],
                 preferred_element_type=jnp.float32)
    h += jnp.dot(r_ref[...], w1[2 * _EMB:3 * _EMB, :],
                 preferred_element_type=jnp.float32)
    h = jnp.maximum(h + b1_ref[...], 0.0)
    h = jnp.dot(h, W2_ref[...], preferred_element_type=jnp.float32)
    h = jnp.maximum(h + b2_ref[...], 0.0)
    out_ref[...] = (jnp.dot(h, W3_ref[...], preferred_element_type=jnp.float32)
                    + b3_ref[...])


def _mlp(g, a, r, W1, b1, W2, b2, W3, b3):
    grid = (_BATCH // _BLK,)
    row_spec = pl.BlockSpec((_BLK, _EMB), lambda i: (i, 0))
    full = lambda shape: pl.BlockSpec(shape, lambda i: (0,) * len(shape))
    return pl.pallas_call(
        _mlp_body,
        grid=grid,
        in_specs=[
            row_spec, row_spec, row_spec,
            full((3 * _EMB, 64)), full((1, 64)),
            full((64, _EMB)), full((1, _EMB)),
            full((_EMB, _EMB)), full((1, _EMB)),
        ],
        out_specs=pl.BlockSpec((_BLK, _EMB), lambda i: (i, 0)),
        out_shape=jax.ShapeDtypeStruct((_BATCH, _EMB), jnp.float32),
    )(g, a, r, W1, b1.reshape(1, -1), W2, b2.reshape(1, -1),
      W3, b3.reshape(1, -1))


def kernel(genre_id, author_id, artist_id, genre_table, author_table,
           artist_table, W1, b1, W2, b2, W3, b3):
    def _tail128(t):
        return t[(t.shape[0] // 128) * 128:].reshape(-1, 128)

    g128, a128, r128 = _sc_pack(genre_table.T, author_table.T,
                                artist_table.T, _tail128(genre_table),
                                _tail128(author_table),
                                _tail128(artist_table))
    g, a, r = _sc_gather(genre_id, author_id, artist_id, g128, a128, r128)
    return _mlp(g, a, r, W1, b1, W2, b2, W3, b3)
